# baseline (speedup 1.0000x reference)
import functools

import jax
import jax.numpy as jnp
from jax import lax
from jax.experimental import pallas as pl
from jax.experimental.pallas import tpu as pltpu
from jax.experimental.pallas import tpu_sc as plsc


def kernel(x, table):
    seq_len = x.shape[1]
    dim = table.shape[1]
    info = plsc.get_sparse_core_info()
    nc, ns = info.num_cores, info.num_subcores
    rows_per_core = seq_len // nc
    rows_per_w = rows_per_core // ns
    half = rows_per_w // 2

    mesh = plsc.VectorSubcoreMesh(core_axis_name="c", subcore_axis_name="s")

    @functools.partial(
        pl.kernel,
        mesh=mesh,
        out_type=jax.ShapeDtypeStruct((seq_len, dim), table.dtype),
        scratch_types=[
            pltpu.VMEM_SHARED((rows_per_core, dim), table.dtype),
            pltpu.SemaphoreType.DMA((2,)),
            pltpu.SemaphoreType.DMA((2,)),
        ],
    )
    def sc_copy(table_hbm, out_hbm, spmem, in_sems, out_sems):
        cid = lax.axis_index("c")
        sid = lax.axis_index("s")
        hbm_base = cid * rows_per_core + sid * rows_per_w
        sp_base = sid * rows_per_w

        def cin(j):
            return pltpu.make_async_copy(
                table_hbm.at[pl.ds(hbm_base + j * half, half), :],
                spmem.at[pl.ds(sp_base + j * half, half), :],
                in_sems.at[j],
            )

        def cout(j):
            return pltpu.make_async_copy(
                spmem.at[pl.ds(sp_base + j * half, half), :],
                out_hbm.at[pl.ds(hbm_base + j * half, half), :],
                out_sems.at[j],
            )

        cin(0).start()
        cin(1).start()
        cin(0).wait()
        cout(0).start()
        cin(1).wait()
        cout(1).start()
        cout(0).wait()
        cout(1).wait()

    return sc_copy(table)


# speedup vs baseline: 1.0030x; 1.0030x over previous
import functools

import jax
import jax.numpy as jnp
from jax import lax
from jax.experimental import pallas as pl
from jax.experimental.pallas import tpu as pltpu
from jax.experimental.pallas import tpu_sc as plsc


def kernel(x, table):
    seq_len = x.shape[1]
    dim = table.shape[1]
    info = plsc.get_sparse_core_info()
    nc, ns = info.num_cores, info.num_subcores
    rows_per_core = seq_len // nc
    rows_per_w = rows_per_core // ns
    nchunks = 4
    half = rows_per_w // nchunks

    mesh = plsc.VectorSubcoreMesh(core_axis_name="c", subcore_axis_name="s")

    @functools.partial(
        pl.kernel,
        mesh=mesh,
        out_type=jax.ShapeDtypeStruct((seq_len, dim), table.dtype),
        scratch_types=[
            pltpu.VMEM_SHARED((rows_per_core, dim), table.dtype),
            pltpu.SemaphoreType.DMA((4,)),
            pltpu.SemaphoreType.DMA((4,)),
        ],
    )
    def sc_copy(table_hbm, out_hbm, spmem, in_sems, out_sems):
        cid = lax.axis_index("c")
        sid = lax.axis_index("s")
        hbm_base = cid * rows_per_core + sid * rows_per_w
        sp_base = sid * rows_per_w

        def cin(j):
            return pltpu.make_async_copy(
                table_hbm.at[pl.ds(hbm_base + j * half, half), :],
                spmem.at[pl.ds(sp_base + j * half, half), :],
                in_sems.at[j],
            )

        def cout(j):
            return pltpu.make_async_copy(
                spmem.at[pl.ds(sp_base + j * half, half), :],
                out_hbm.at[pl.ds(hbm_base + j * half, half), :],
                out_sems.at[j],
            )

        for j in range(nchunks):
            cin(j).start()
        for j in range(nchunks):
            cin(j).wait()
            cout(j).start()
        for j in range(nchunks):
            cout(j).wait()

    return sc_copy(table)


# speedup vs baseline: 3.0455x; 3.0364x over previous
import jax
import jax.numpy as jnp
from jax.experimental import pallas as pl

_BLOCK_ROWS = 2048


def _copy_body(t_ref, o_ref):
    o_ref[...] = t_ref[...]


def kernel(x, table):
    seq_len = x.shape[1]
    dim = table.shape[1]
    return pl.pallas_call(
        _copy_body,
        grid=(seq_len // _BLOCK_ROWS,),
        in_specs=[pl.BlockSpec((_BLOCK_ROWS, dim), lambda i: (i, 0))],
        out_specs=pl.BlockSpec((_BLOCK_ROWS, dim), lambda i: (i, 0)),
        out_shape=jax.ShapeDtypeStruct((seq_len, dim), table.dtype),
    )(table)
